# baseline (speedup 1.0000x reference)
import functools

import jax
import jax.numpy as jnp
from jax import lax
from jax.experimental import pallas as pl
from jax.experimental.pallas import tpu as pltpu
from jax.experimental.pallas import tpu_sc as plsc

_NUM_CLASSES = 100000
_FEAT = 64
_BATCH = 16384
_NC = 2
_NS = 16
_L = 16
_NW = _NC * _NS
_ROWS_W = _BATCH // _NW
_ROWS_S = _BATCH // _NS
_HIST_PAD = 100352
_HCHUNK = _HIST_PAD // _NS


def _sc_body(x_hbm, lbl_hbm, cen_hbm, zeros_hbm, ones_hbm, out_hbm,
             lbl_v, ones_v, xv, cv, cnt_v, acc_v, hist, sem_c, sem_x):
    c = lax.axis_index("c")
    s = lax.axis_index("s")
    pltpu.sync_copy(lbl_hbm.at[pl.ds(s * 8, 8)], lbl_v)
    pltpu.sync_copy(ones_hbm, ones_v)
    pltpu.sync_copy(zeros_hbm.at[pl.ds(s * _HCHUNK, _HCHUNK)],
                    hist.at[pl.ds(s * _HCHUNK, _HCHUNK)])
    row0 = s * _ROWS_S + c * _ROWS_W
    cp_x = pltpu.async_copy(x_hbm.at[pl.ds(row0, _ROWS_W)], xv, sem_x)
    cps = []
    for j in range(4):
        cps.append(pltpu.async_copy(
            cen_hbm.at[lbl_v.at[c * 4 + j]],
            cv.at[pl.ds(j * 128, 128)], sem_c))
    plsc.subcore_barrier()
    for j in range(8):
        pltpu.sync_copy(ones_v, hist.at[lbl_v.at[j]], add=True)
    plsc.subcore_barrier()
    for j in range(4):
        pltpu.sync_copy(hist.at[lbl_v.at[c * 4 + j]],
                        cnt_v.at[pl.ds(j * 128, 128)])
    cp_x.wait()
    for cp in cps:
        cp.wait()

    def dbody(g, acc):
        cnt = cnt_v[pl.ds(g * _L, _L)]
        wblk = 0.5 / (cnt + 1.0)
        for j in range(_L):
            r = g * _L + j
            ssq = None
            for k in range(_FEAT // _L):
                d = xv[r, pl.ds(k * _L, _L)] - cv[r, pl.ds(k * _L, _L)]
                ssq = d * d if ssq is None else ssq + d * d
            acc = acc + lax.broadcast(wblk[j], (_L,)) * ssq
        return acc

    acc = lax.fori_loop(0, _ROWS_W // _L, dbody,
                        jnp.zeros((_L,), jnp.float32))
    acc_v[...] = acc
    pltpu.sync_copy(acc_v, out_hbm.at[s * _NC + c])


_sc_call = functools.partial(
    pl.kernel,
    mesh=plsc.VectorSubcoreMesh(core_axis_name="c", subcore_axis_name="s"),
    out_type=jax.ShapeDtypeStruct((_NW, _L), jnp.float32),
    compiler_params=pltpu.CompilerParams(use_tc_tiling_on_sc=False),
    scratch_types=[
        pltpu.VMEM((8, 128), jnp.int32),
        pltpu.VMEM((128,), jnp.float32),
        pltpu.VMEM((_ROWS_W, _FEAT), jnp.float32),
        pltpu.VMEM((_ROWS_W, _FEAT), jnp.float32),
        pltpu.VMEM((_ROWS_W,), jnp.float32),
        pltpu.VMEM((_L,), jnp.float32),
        pltpu.VMEM_SHARED((_HIST_PAD,), jnp.float32),
        pltpu.SemaphoreType.DMA,
        pltpu.SemaphoreType.DMA,
    ],
)(_sc_body)


def kernel(x, labels, center):
    lbl = labels.astype(jnp.int32).reshape(128, 128)
    zeros = jnp.zeros((_HIST_PAD,), jnp.float32)
    ones = jnp.ones((128,), jnp.float32)
    out = _sc_call(x, lbl, center, zeros, ones)
    return jnp.sum(out)
